# Initial kernel scaffold; baseline (speedup 1.0000x reference)
import jax
import jax.numpy as jnp
from jax.experimental import pallas as pl


def kernel(seq, token_table, pos_table):
    raise NotImplementedError("write your pallas kernel here")


# speedup vs baseline: 3.1210x; 3.1210x over previous
import functools

import jax
import jax.numpy as jnp
from jax import lax
from jax.experimental import pallas as pl
from jax.experimental.pallas import tpu as pltpu
from jax.experimental.pallas import tpu_sc as plsc

NC, NS = 2, 16
NW = NC * NS
B, L, D = 1024, 200, 64
VECL = 16
SEQ_PER_W = B // NW
HALF = L // 2

_mesh = plsc.VectorSubcoreMesh(core_axis_name="c", subcore_axis_name="s")


@functools.partial(
    pl.kernel,
    out_type=jax.ShapeDtypeStruct((B * L, D), jnp.float32),
    mesh=_mesh,
    compiler_params=pltpu.CompilerParams(use_tc_tiling_on_sc=False),
    scratch_types=[
        pltpu.VMEM((SEQ_PER_W * 2, HALF), jnp.int32),
        pltpu.VMEM((L, D), jnp.float32),
        pltpu.VMEM((2, L, D), jnp.float32),
        pltpu.SemaphoreType.DMA((2,)),
        pltpu.SemaphoreType.DMA((2,)),
    ],
)
def _seq_embed(seq_hbm, tok_hbm, pos_hbm, out_hbm, idx_v, pos_v, buf, gsem, ssem):
    wid = lax.axis_index("s") * NC + lax.axis_index("c")
    idx_row0 = wid * (SEQ_PER_W * 2)
    out_seq0 = wid * SEQ_PER_W

    pltpu.sync_copy(seq_hbm.at[pl.ds(idx_row0, SEQ_PER_W * 2)], idx_v)
    pltpu.sync_copy(pos_hbm, pos_v)

    def gather_start(i, b):
        pltpu.async_copy(tok_hbm.at[idx_v.at[2 * i]],
                         buf.at[b, pl.ds(0, HALF)], gsem.at[b])
        pltpu.async_copy(tok_hbm.at[idx_v.at[2 * i + 1]],
                         buf.at[b, pl.ds(HALF, HALF)], gsem.at[b])

    def gather_wait(b):
        pltpu.make_async_copy(out_hbm.at[pl.ds(0, L)], buf.at[b], gsem.at[b]).wait()

    def scatter_start(i, b):
        pltpu.async_copy(buf.at[b], out_hbm.at[pl.ds((out_seq0 + i) * L, L)],
                         ssem.at[b])

    def scatter_wait(b):
        pltpu.make_async_copy(buf.at[b], out_hbm.at[pl.ds(0, L)], ssem.at[b]).wait()

    def add_pos(b):
        @plsc.parallel_loop(0, L, unroll=4)
        def _(r):
            for j in range(D // VECL):
                s = pl.ds(j * VECL, VECL)
                buf[b, r, s] = buf[b, r, s] + pos_v[r, s]

    gather_start(0, 0)

    @pl.loop(0, SEQ_PER_W, step=2)
    def _(ii):
        for b in range(2):
            i = ii + b
            gather_wait(b)

            @pl.when(i >= 1)
            def _():
                scatter_wait(1 - b)

            @pl.when(i + 1 < SEQ_PER_W)
            def _():
                gather_start(i + 1, 1 - b)

            add_pos(b)
            scatter_start(i, b)

    scatter_wait(1)


def kernel(seq, token_table, pos_table):
    seq2 = seq.reshape(B * L // HALF, HALF).astype(jnp.int32)
    out = _seq_embed(seq2, token_table, pos_table)
    return out.reshape(B, L, D)


# speedup vs baseline: 3.1252x; 1.0013x over previous
import functools

import jax
import jax.numpy as jnp
from jax import lax
from jax.experimental import pallas as pl
from jax.experimental.pallas import tpu as pltpu
from jax.experimental.pallas import tpu_sc as plsc

NC, NS = 2, 16
NW = NC * NS
B, L, D = 1024, 200, 64
VECL = 16
SEQ_PER_W = B // NW
S0, S1 = 128, L - 128

_mesh = plsc.VectorSubcoreMesh(core_axis_name="c", subcore_axis_name="s")


@functools.partial(
    pl.kernel,
    out_type=jax.ShapeDtypeStruct((B, L, D), jnp.float32),
    mesh=_mesh,
    compiler_params=pltpu.CompilerParams(use_tc_tiling_on_sc=False),
    scratch_types=[
        pltpu.VMEM((SEQ_PER_W, L), jnp.int32),
        pltpu.VMEM((L, D), jnp.float32),
        pltpu.VMEM((2, L, D), jnp.float32),
        pltpu.SemaphoreType.DMA((2,)),
        pltpu.SemaphoreType.DMA((2,)),
    ],
)
def _seq_embed(seq_hbm, tok_hbm, pos_hbm, out_hbm, idx_v, pos_v, buf, gsem, ssem):
    wid = lax.axis_index("s") * NC + lax.axis_index("c")
    seq0 = wid * SEQ_PER_W

    pltpu.sync_copy(seq_hbm.at[pl.ds(seq0, SEQ_PER_W)], idx_v)
    pltpu.sync_copy(pos_hbm, pos_v)

    def gather_start(i, b):
        pltpu.async_copy(tok_hbm.at[idx_v.at[i, pl.ds(0, S0)]],
                         buf.at[b, pl.ds(0, S0)], gsem.at[b])
        pltpu.async_copy(tok_hbm.at[idx_v.at[i, pl.ds(S0, S1)]],
                         buf.at[b, pl.ds(S0, S1)], gsem.at[b])

    def gather_wait(b):
        pltpu.make_async_copy(out_hbm.at[0], buf.at[b], gsem.at[b]).wait()

    def scatter_start(i, b):
        pltpu.async_copy(buf.at[b], out_hbm.at[seq0 + i], ssem.at[b])

    def scatter_wait(b):
        pltpu.make_async_copy(buf.at[b], out_hbm.at[0], ssem.at[b]).wait()

    def add_pos(b):
        @plsc.parallel_loop(0, L, unroll=4)
        def _(r):
            for j in range(D // VECL):
                s = pl.ds(j * VECL, VECL)
                buf[b, r, s] = buf[b, r, s] + pos_v[r, s]

    gather_start(0, 0)

    @pl.loop(0, SEQ_PER_W, step=2)
    def _(ii):
        for b in range(2):
            i = ii + b
            gather_wait(b)

            @pl.when(i >= 1)
            def _():
                scatter_wait(1 - b)

            @pl.when(i + 1 < SEQ_PER_W)
            def _():
                gather_start(i + 1, 1 - b)

            add_pos(b)
            scatter_start(i, b)

    scatter_wait(1)


def kernel(seq, token_table, pos_table):
    return _seq_embed(seq, token_table, pos_table)


# speedup vs baseline: 3.1491x; 1.0076x over previous
import functools

import jax
import jax.numpy as jnp
from jax import lax
from jax.experimental import pallas as pl
from jax.experimental.pallas import tpu as pltpu
from jax.experimental.pallas import tpu_sc as plsc

NC, NS = 2, 16
NW = NC * NS
B, L, D = 1024, 200, 64
VECL = 16
BB = 128
NB = B // BB
L_PER_W = L // (NW // NB)

_mesh = plsc.VectorSubcoreMesh(core_axis_name="c", subcore_axis_name="s")


@functools.partial(
    pl.kernel,
    out_type=jax.ShapeDtypeStruct((L, D // 8, NB, 8, BB), jnp.float32),
    mesh=_mesh,
    compiler_params=pltpu.CompilerParams(use_tc_tiling_on_sc=False,
                                        needs_layout_passes=False),
    scratch_types=[
        pltpu.VMEM((L_PER_W, BB), jnp.int32),
        pltpu.VMEM((L_PER_W * D,), jnp.float32),
        pltpu.VMEM((2, BB, D), jnp.float32),
        pltpu.VMEM((2, D // 8, 8, BB), jnp.float32),
        pltpu.SemaphoreType.DMA((2,)),
        pltpu.SemaphoreType.DMA((2,)),
    ],
)
def _seq_embed(seq_t_hbm, tok_hbm, pos_hbm, out_hbm, idx_v, pos_v, gbuf, tbuf,
               gsem, ssem):
    wid = lax.axis_index("s") * NC + lax.axis_index("c")
    l0 = (wid // NB) * L_PER_W
    bhi = wid % NB

    pltpu.sync_copy(seq_t_hbm.at[pl.ds(l0, L_PER_W), pl.ds(bhi * BB, BB)], idx_v)
    pltpu.sync_copy(pos_hbm.at[pl.ds(l0 * D, L_PER_W * D)], pos_v)

    def gather_start(u, b):
        pltpu.async_copy(tok_hbm.at[idx_v.at[u]], gbuf.at[b], gsem.at[b])

    def gather_wait(b):
        pltpu.make_async_copy(tok_hbm.at[pl.ds(0, BB)], gbuf.at[b],
                              gsem.at[b]).wait()

    def scatter_start(u, b):
        pltpu.async_copy(tbuf.at[b], out_hbm.at[l0 + u, :, bhi], ssem.at[b])

    def scatter_wait(b):
        pltpu.make_async_copy(tbuf.at[b], out_hbm.at[0, :, 0], ssem.at[b]).wait()

    lane_ids = jnp.arange(VECL, dtype=jnp.int32)

    def transpose_add(u, b):
        @plsc.parallel_loop(0, D, unroll=2)
        def _(d):
            col = jnp.full((VECL,), d, jnp.int32)
            p = plsc.load_gather(pos_v, [u * D + col])
            for g in range(BB // VECL):
                rows = lane_ids + (g * VECL)
                val = plsc.load_gather(gbuf.at[b], [rows, col])
                tbuf[b, d // 8, d % 8, pl.ds(g * VECL, VECL)] = val + p

    gather_start(0, 0)

    @pl.loop(0, L_PER_W, step=2)
    def _(uu):
        for b in range(2):
            u = uu + b

            gather_wait(b)

            @pl.when(u + 1 < L_PER_W)
            def _():
                gather_start(u + 1, 1 - b)

            @pl.when(u >= 2)
            def _():
                scatter_wait(b)

            transpose_add(u, b)
            scatter_start(u, b)

    scatter_wait(0)
    scatter_wait(1)


def kernel(seq, token_table, pos_table):
    out5 = _seq_embed(seq.T, token_table, pos_table.reshape(-1))
    return jnp.transpose(out5, (2, 4, 0, 1, 3)).reshape(B, L, D)
